# baseline (speedup 1.0000x reference)
import functools

import jax
import jax.numpy as jnp
from jax import lax
from jax.experimental import pallas as pl
from jax.experimental.pallas import tpu as pltpu
from jax.experimental.pallas import tpu_sc as plsc

VOCAB = 1000000
EMB = 64

NC = 2
NS = 16
NW = NC * NS

C = 128


def _make_gather(B: int, D: int):
    assert B % (NW * C) == 0
    nsteps = B // (NW * C)
    assert nsteps % 2 == 0
    nrounds = nsteps // 2
    b_per_w = B // NW

    mesh = plsc.VectorSubcoreMesh(core_axis_name="c", subcore_axis_name="s")

    @functools.partial(
        pl.kernel,
        mesh=mesh,
        out_type=jax.ShapeDtypeStruct((B, D), jnp.float32),
        compiler_params=pltpu.CompilerParams(use_tc_tiling_on_sc=False),
        scratch_types=[
            pltpu.VMEM((nsteps, C), jnp.int32),
            pltpu.VMEM((C, D), jnp.float32),
            pltpu.VMEM((C, D), jnp.float32),
            pltpu.SemaphoreType.DMA,
            pltpu.SemaphoreType.DMA,
        ],
    )
    def gather_kernel(idx_hbm, table_hbm, out_hbm, idx_v, buf0, buf1,
                      sem0, sem1):
        wid = lax.axis_index("c") * NS + lax.axis_index("s")
        row0 = wid * nsteps
        out0 = wid * b_per_w

        pltpu.sync_copy(idx_hbm.at[pl.ds(row0, nsteps)], idx_v)

        def start(g, buf, sem):
            pltpu.async_copy(table_hbm.at[idx_v.at[g]], buf, sem)

        def wait(g, buf, sem):
            pltpu.make_async_copy(table_hbm.at[idx_v.at[g]], buf, sem).wait()

        def write(g, buf):
            pltpu.sync_copy(buf, out_hbm.at[pl.ds(out0 + g * C, C)])

        start(0, buf0, sem0)
        start(1, buf1, sem1)

        def round_body(r, _):
            g0 = r * 2
            g1 = g0 + 1
            wait(g0, buf0, sem0)
            write(g0, buf0)
            start(g0 + 2, buf0, sem0)
            wait(g1, buf1, sem1)
            write(g1, buf1)
            start(g1 + 2, buf1, sem1)
            return 0

        lax.fori_loop(0, nrounds - 1, round_body, 0)

        g0 = (nrounds - 1) * 2
        wait(g0, buf0, sem0)
        write(g0, buf0)
        wait(g0 + 1, buf1, sem1)
        write(g0 + 1, buf1)

    return gather_kernel


def kernel(inputs, weight):
    n, s = inputs.shape
    B = n * s
    idx = inputs.reshape(B // C, C).astype(jnp.int32)
    out = _make_gather(B, EMB)(idx, weight)
    return out.reshape(n, s, EMB)


# speedup vs baseline: 1.0234x; 1.0234x over previous
import functools

import jax
import jax.numpy as jnp
from jax import lax
from jax.experimental import pallas as pl
from jax.experimental.pallas import tpu as pltpu
from jax.experimental.pallas import tpu_sc as plsc

VOCAB = 1000000
EMB = 64

NC = 2
NS = 16
NW = NC * NS

C = 128
K = 4
BR = C * K


def _make_gather(B: int, D: int):
    assert B % (NW * BR) == 0
    nblocks = B // (NW * BR)
    assert nblocks % 2 == 0
    nrounds = nblocks // 2
    nsteps = nblocks * K
    b_per_w = B // NW

    mesh = plsc.VectorSubcoreMesh(core_axis_name="c", subcore_axis_name="s")

    @functools.partial(
        pl.kernel,
        mesh=mesh,
        out_type=jax.ShapeDtypeStruct((B, D), jnp.float32),
        compiler_params=pltpu.CompilerParams(use_tc_tiling_on_sc=False),
        scratch_types=[
            pltpu.VMEM((nsteps, C), jnp.int32),
            pltpu.VMEM((BR, D), jnp.float32),
            pltpu.VMEM((BR, D), jnp.float32),
            pltpu.SemaphoreType.DMA,
            pltpu.SemaphoreType.DMA,
        ],
    )
    def gather_kernel(idx_hbm, table_hbm, out_hbm, idx_v, buf0, buf1,
                      sem0, sem1):
        wid = lax.axis_index("c") * NS + lax.axis_index("s")
        row0 = wid * nsteps
        out0 = wid * b_per_w

        pltpu.sync_copy(idx_hbm.at[pl.ds(row0, nsteps)], idx_v)

        def fire(g, buf, sem):
            for j in range(K):
                pltpu.async_copy(table_hbm.at[idx_v.at[g * K + j]],
                                 buf.at[pl.ds(j * C, C)], sem)

        def drain(buf, sem):
            pltpu.make_async_copy(table_hbm.at[pl.ds(0, BR)], buf, sem).wait()

        def write(g, buf):
            pltpu.sync_copy(buf, out_hbm.at[pl.ds(out0 + g * BR, BR)])

        fire(0, buf0, sem0)
        fire(1, buf1, sem1)

        def round_body(r, _):
            g0 = r * 2
            g1 = g0 + 1
            drain(buf0, sem0)
            write(g0, buf0)
            fire(g0 + 2, buf0, sem0)
            drain(buf1, sem1)
            write(g1, buf1)
            fire(g1 + 2, buf1, sem1)
            return 0

        lax.fori_loop(0, nrounds - 1, round_body, 0)

        g0 = (nrounds - 1) * 2
        drain(buf0, sem0)
        write(g0, buf0)
        drain(buf1, sem1)
        write(g0 + 1, buf1)

    return gather_kernel


def kernel(inputs, weight):
    n, s = inputs.shape
    B = n * s
    idx = inputs.reshape(B // C, C).astype(jnp.int32)
    out = _make_gather(B, EMB)(idx, weight)
    return out.reshape(n, s, EMB)
